# baseline (speedup 1.0000x reference)
import functools

import jax
import jax.numpy as jnp
from jax.experimental import pallas as pl
from jax.experimental.pallas import tpu as pltpu

_BINS = 64
_EPS = 1e-10
_SUB_R = 16


def _loss_kernel(pa_ref, ta_ref, f_ref, w_ref, out_ref, mm_ref, acc_ref,
                 *, nchunk, chunk_r):
    phase = pl.program_id(1)
    c = pl.program_id(2)

    @pl.when(jnp.logical_and(phase == 0, c == 0))
    def _init_mm():
        mm_ref[0] = jnp.float32(jnp.inf)
        mm_ref[1] = jnp.float32(-jnp.inf)
        mm_ref[2] = jnp.float32(jnp.inf)
        mm_ref[3] = jnp.float32(-jnp.inf)

    @pl.when(phase == 0)
    def _minmax():
        fb = f_ref[0]
        wb = w_ref[0]
        mm_ref[0] = jnp.minimum(mm_ref[0], jnp.min(fb))
        mm_ref[1] = jnp.maximum(mm_ref[1], jnp.max(fb))
        mm_ref[2] = jnp.minimum(mm_ref[2], jnp.min(wb))
        mm_ref[3] = jnp.maximum(mm_ref[3], jnp.max(wb))

    @pl.when(phase == 1)
    def _hist():
        @pl.when(c == 0)
        def _zero_acc():
            acc_ref[...] = jnp.zeros_like(acc_ref)

        fmin = mm_ref[0]
        wmin = mm_ref[2]
        inv_f = 1.0 / (mm_ref[1] - fmin + _EPS)
        inv_w = 1.0 / (mm_ref[3] - wmin + _EPS)

        lanes = f_ref.shape[2]
        bins2d = jax.lax.broadcasted_iota(
            jnp.int32, (_BINS, lanes), 0).astype(jnp.int8)
        f8t = jnp.float8_e4m3fn

        def body(i, _):
            f8 = f_ref[0, pl.ds(i * _SUB_R, _SUB_R), :]
            w8 = w_ref[0, pl.ds(i * _SUB_R, _SUB_R), :]
            fi8 = jnp.clip(jnp.floor((f8 - fmin) * inv_f * (_BINS - 1)),
                           0.0, _BINS - 1.0).astype(jnp.int8)
            wi8 = jnp.clip(jnp.floor((w8 - wmin) * inv_w * (_BINS - 1)),
                           0.0, _BINS - 1.0).astype(jnp.int8)
            acc = acc_ref[...]
            for j in range(_SUB_R):
                fr = jax.lax.slice(fi8, (j, 0), (j + 1, lanes))
                wr = jax.lax.slice(wi8, (j, 0), (j + 1, lanes))
                ohf = jnp.where(bins2d == fr, f8t(1), f8t(0))
                ohw = jnp.where(bins2d == wr, f8t(1), f8t(0))
                acc = jax.lax.dot_general(
                    ohf, ohw, (((1,), (1,)), ((), ())),
                    preferred_element_type=jnp.float32) + acc
            acc_ref[...] = acc
            return 0

        jax.lax.fori_loop(0, chunk_r // _SUB_R, body, 0)

        @pl.when(c == nchunk - 1)
        def _finalize():
            counts = acc_ref[...]
            total = jnp.sum(counts)
            inv_n = 1.0 / (total + _EPS)
            joint = counts * inv_n
            p_f = jnp.sum(joint, axis=1, keepdims=True)
            p_w = jnp.sum(joint, axis=0, keepdims=True)
            h_f = -jnp.sum(p_f * jnp.log(p_f + _EPS))
            h_w = -jnp.sum(p_w * jnp.log(p_w + _EPS))
            h_j = -jnp.sum(joint * jnp.log(joint + _EPS))
            mi = h_f + h_w - h_j
            nmi = 2.0 * mi / (h_f + h_w + _EPS)
            d = pa_ref[0] - ta_ref[0]
            sumsq = jnp.sum(d * d)
            lane = jax.lax.broadcasted_iota(jnp.int32, (1, 128), 1)
            out_ref[0] = jnp.where(lane == 0, sumsq,
                                   jnp.where(lane == 1, nmi, 0.0))


def _choose_chunk(nr):
    for cr in (336, 256, 128, 64, 32, 16, 8):
        if nr % cr == 0:
            return cr
    return nr


@jax.jit
def kernel(predicted_affine, true_affine, fixed, warped):
    b = fixed.shape[0]
    lanes = fixed.shape[4]
    nr = fixed.shape[2] * fixed.shape[3]
    chunk_r = _choose_chunk(nr)
    nchunk = nr // chunk_r

    f = fixed[:, 0].reshape(b, nr, lanes)
    w = warped[:, 0].reshape(b, nr, lanes)
    na = predicted_affine.shape[1]
    pa = jnp.pad(predicted_affine[:, None, :], ((0, 0), (0, 0), (0, 128 - na)))
    ta = jnp.pad(true_affine[:, None, :], ((0, 0), (0, 0), (0, 128 - na)))

    out = pl.pallas_call(
        functools.partial(_loss_kernel, nchunk=nchunk, chunk_r=chunk_r),
        grid=(b, 2, nchunk),
        in_specs=[
            pl.BlockSpec((1, 1, 128), lambda bb, p, c: (bb, 0, 0)),
            pl.BlockSpec((1, 1, 128), lambda bb, p, c: (bb, 0, 0)),
            pl.BlockSpec((1, chunk_r, lanes), lambda bb, p, c: (bb, c, 0)),
            pl.BlockSpec((1, chunk_r, lanes), lambda bb, p, c: (bb, c, 0)),
        ],
        out_specs=pl.BlockSpec((1, 1, 128), lambda bb, p, c: (bb, 0, 0)),
        out_shape=jax.ShapeDtypeStruct((b, 1, 128), jnp.float32),
        scratch_shapes=[
            pltpu.SMEM((4,), jnp.float32),
            pltpu.VMEM((_BINS, _BINS), jnp.float32),
        ],
        compiler_params=pltpu.CompilerParams(
            dimension_semantics=("parallel", "arbitrary", "arbitrary")),
    )(pa, ta, f, w)

    affine_loss = jnp.sum(out[:, 0, 0]) / (b * na)
    sim_loss = -jnp.mean(out[:, 0, 1])
    return affine_loss + sim_loss


# speedup vs baseline: 1.3712x; 1.3712x over previous
import functools

import jax
import jax.numpy as jnp
from jax.experimental import pallas as pl
from jax.experimental.pallas import tpu as pltpu

_BINS = 64
_EPS = 1e-10
_SUB_R = 16


def _loss_kernel(pa_ref, ta_ref, f_ref, w_ref, out_ref, mm_ref, acc_ref,
                 *, nchunk):
    phase = pl.program_id(1)
    c = pl.program_id(2)

    @pl.when(jnp.logical_and(phase == 0, c == 0))
    def _init_mm():
        mm_ref[0] = jnp.float32(jnp.inf)
        mm_ref[1] = jnp.float32(-jnp.inf)
        mm_ref[2] = jnp.float32(jnp.inf)
        mm_ref[3] = jnp.float32(-jnp.inf)

    @pl.when(phase == 0)
    def _minmax():
        fb = f_ref[0, 0]
        wb = w_ref[0, 0]
        mm_ref[0] = jnp.minimum(mm_ref[0], jnp.min(fb))
        mm_ref[1] = jnp.maximum(mm_ref[1], jnp.max(fb))
        mm_ref[2] = jnp.minimum(mm_ref[2], jnp.min(wb))
        mm_ref[3] = jnp.maximum(mm_ref[3], jnp.max(wb))

    @pl.when(phase == 1)
    def _hist():
        @pl.when(c == 0)
        def _zero_acc():
            acc_ref[...] = jnp.zeros_like(acc_ref)

        fmin = mm_ref[0]
        wmin = mm_ref[2]
        inv_f = 1.0 / (mm_ref[1] - fmin + _EPS)
        inv_w = 1.0 / (mm_ref[3] - wmin + _EPS)

        bd, hh, lanes = f_ref.shape[2], f_ref.shape[3], f_ref.shape[4]
        bins2d = jax.lax.broadcasted_iota(
            jnp.int32, (_BINS, lanes), 0).astype(jnp.int8)
        f8t = jnp.float8_e4m3fn

        def make_body(dd):
            def body(i, _):
                f8 = f_ref[0, 0, dd, pl.ds(i * _SUB_R, _SUB_R), :]
                w8 = w_ref[0, 0, dd, pl.ds(i * _SUB_R, _SUB_R), :]
                fi8 = jnp.clip(jnp.floor((f8 - fmin) * inv_f * (_BINS - 1)),
                               0.0, _BINS - 1.0).astype(jnp.int8)
                wi8 = jnp.clip(jnp.floor((w8 - wmin) * inv_w * (_BINS - 1)),
                               0.0, _BINS - 1.0).astype(jnp.int8)
                acc = acc_ref[...]
                for j in range(_SUB_R):
                    fr = jax.lax.slice(fi8, (j, 0), (j + 1, lanes))
                    wr = jax.lax.slice(wi8, (j, 0), (j + 1, lanes))
                    ohf = jnp.where(bins2d == fr, f8t(1), f8t(0))
                    ohw = jnp.where(bins2d == wr, f8t(1), f8t(0))
                    acc = jax.lax.dot_general(
                        ohf, ohw, (((1,), (1,)), ((), ())),
                        preferred_element_type=jnp.float32) + acc
                acc_ref[...] = acc
                return 0
            return body

        for dd in range(bd):
            jax.lax.fori_loop(0, hh // _SUB_R, make_body(dd), 0)

        @pl.when(c == nchunk - 1)
        def _finalize():
            counts = acc_ref[...]
            total = jnp.sum(counts)
            inv_n = 1.0 / (total + _EPS)
            joint = counts * inv_n
            p_f = jnp.sum(joint, axis=1, keepdims=True)
            p_w = jnp.sum(joint, axis=0, keepdims=True)
            h_f = -jnp.sum(p_f * jnp.log(p_f + _EPS))
            h_w = -jnp.sum(p_w * jnp.log(p_w + _EPS))
            h_j = -jnp.sum(joint * jnp.log(joint + _EPS))
            mi = h_f + h_w - h_j
            nmi = 2.0 * mi / (h_f + h_w + _EPS)
            d = pa_ref[0] - ta_ref[0]
            sumsq = jnp.sum(d * d)
            lane = jax.lax.broadcasted_iota(jnp.int32, (1, 128), 1)
            out_ref[0] = jnp.where(lane == 0, sumsq,
                                   jnp.where(lane == 1, nmi, 0.0))


def _choose_bd(d, h):
    for bd in (3, 2, 1):
        if d % bd == 0 and h % _SUB_R == 0:
            return bd
    return 1


@jax.jit
def kernel(predicted_affine, true_affine, fixed, warped):
    b = fixed.shape[0]
    d, h, lanes = fixed.shape[2], fixed.shape[3], fixed.shape[4]
    bd = _choose_bd(d, h)
    nchunk = d // bd

    na = predicted_affine.shape[1]
    pa = jnp.pad(predicted_affine[:, None, :], ((0, 0), (0, 0), (0, 128 - na)))
    ta = jnp.pad(true_affine[:, None, :], ((0, 0), (0, 0), (0, 128 - na)))

    out = pl.pallas_call(
        functools.partial(_loss_kernel, nchunk=nchunk),
        grid=(b, 2, nchunk),
        in_specs=[
            pl.BlockSpec((1, 1, 128), lambda bb, p, c: (bb, 0, 0)),
            pl.BlockSpec((1, 1, 128), lambda bb, p, c: (bb, 0, 0)),
            pl.BlockSpec((1, 1, bd, h, lanes),
                         lambda bb, p, c: (bb, 0, c, 0, 0)),
            pl.BlockSpec((1, 1, bd, h, lanes),
                         lambda bb, p, c: (bb, 0, c, 0, 0)),
        ],
        out_specs=pl.BlockSpec((1, 1, 128), lambda bb, p, c: (bb, 0, 0)),
        out_shape=jax.ShapeDtypeStruct((b, 1, 128), jnp.float32),
        scratch_shapes=[
            pltpu.SMEM((4,), jnp.float32),
            pltpu.VMEM((_BINS, _BINS), jnp.float32),
        ],
        compiler_params=pltpu.CompilerParams(
            dimension_semantics=("parallel", "arbitrary", "arbitrary")),
    )(pa, ta, fixed, warped)

    affine_loss = jnp.sum(out[:, 0, 0]) / (b * na)
    sim_loss = -jnp.mean(out[:, 0, 1])
    return affine_loss + sim_loss


# speedup vs baseline: 1.8996x; 1.3853x over previous
import functools

import jax
import jax.numpy as jnp
from jax.experimental import pallas as pl
from jax.experimental.pallas import tpu as pltpu

_BINS = 64
_EPS = 1e-10
_SUB_R = 32


def _loss_kernel(pa_ref, ta_ref, f_ref, w_ref, out_ref, mm_ref, acc_ref,
                 *, nchunk):
    phase = pl.program_id(1)
    c = pl.program_id(2)

    @pl.when(jnp.logical_and(phase == 0, c == 0))
    def _init_mm():
        mm_ref[0] = jnp.float32(jnp.inf)
        mm_ref[1] = jnp.float32(-jnp.inf)
        mm_ref[2] = jnp.float32(jnp.inf)
        mm_ref[3] = jnp.float32(-jnp.inf)

    @pl.when(phase == 0)
    def _minmax():
        fb = f_ref[0, 0]
        wb = w_ref[0, 0]
        mm_ref[0] = jnp.minimum(mm_ref[0], jnp.min(fb))
        mm_ref[1] = jnp.maximum(mm_ref[1], jnp.max(fb))
        mm_ref[2] = jnp.minimum(mm_ref[2], jnp.min(wb))
        mm_ref[3] = jnp.maximum(mm_ref[3], jnp.max(wb))

    @pl.when(phase == 1)
    def _hist():
        @pl.when(c == 0)
        def _zero_acc():
            acc_ref[...] = jnp.zeros_like(acc_ref)


        fmin = mm_ref[0]
        wmin = mm_ref[2]
        inv_f = 1.0 / (mm_ref[1] - fmin + _EPS)
        inv_w = 1.0 / (mm_ref[3] - wmin + _EPS)

        bd, hh, lanes = f_ref.shape[2], f_ref.shape[3], f_ref.shape[4]
        bins2d = jax.lax.broadcasted_iota(
            jnp.int32, (_BINS, lanes), 0).astype(jnp.int8)
        f8t = jnp.float8_e4m3fn

        def make_body(dd):
            def body(i, _):
                f8 = f_ref[0, 0, dd, pl.ds(i * _SUB_R, _SUB_R), :]
                w8 = w_ref[0, 0, dd, pl.ds(i * _SUB_R, _SUB_R), :]
                fi8 = jnp.clip(jnp.floor((f8 - fmin) * inv_f * (_BINS - 1)),
                               0.0, _BINS - 1.0).astype(jnp.int8)
                wi8 = jnp.clip(jnp.floor((w8 - wmin) * inv_w * (_BINS - 1)),
                               0.0, _BINS - 1.0).astype(jnp.int8)
                acc0 = acc_ref[0]
                acc1 = acc_ref[1]
                for j in range(_SUB_R):
                    fr = jax.lax.slice(fi8, (j, 0), (j + 1, lanes))
                    wr = jax.lax.slice(wi8, (j, 0), (j + 1, lanes))
                    ohf = jnp.where(bins2d == fr, f8t(1), f8t(0))
                    ohw = jnp.where(bins2d == wr, f8t(1), f8t(0))
                    dot = jax.lax.dot_general(
                        ohf, ohw, (((1,), (1,)), ((), ())),
                        preferred_element_type=jnp.float32)
                    if j % 2 == 0:
                        acc0 = dot + acc0
                    else:
                        acc1 = dot + acc1
                acc_ref[0] = acc0
                acc_ref[1] = acc1
                return 0
            return body

        for dd in range(bd):
            jax.lax.fori_loop(0, hh // _SUB_R, make_body(dd), 0)

        @pl.when(c == nchunk - 1)
        def _finalize():
            counts = acc_ref[0] + acc_ref[1]
            total = jnp.sum(counts)
            inv_n = 1.0 / (total + _EPS)
            joint = counts * inv_n
            p_f = jnp.sum(joint, axis=1, keepdims=True)
            p_w = jnp.sum(joint, axis=0, keepdims=True)
            h_f = -jnp.sum(p_f * jnp.log(p_f + _EPS))
            h_w = -jnp.sum(p_w * jnp.log(p_w + _EPS))
            h_j = -jnp.sum(joint * jnp.log(joint + _EPS))
            mi = h_f + h_w - h_j
            nmi = 2.0 * mi / (h_f + h_w + _EPS)
            d = pa_ref[0] - ta_ref[0]
            sumsq = jnp.sum(d * d)
            lane = jax.lax.broadcasted_iota(jnp.int32, (1, 128), 1)
            out_ref[0] = jnp.where(lane == 0, sumsq,
                                   jnp.where(lane == 1, nmi, 0.0))


def _choose_bd(d, h):
    for bd in (6, 4, 3, 2, 1):
        if d % bd == 0 and h % _SUB_R == 0:
            return bd
    return 1


@jax.jit
def kernel(predicted_affine, true_affine, fixed, warped):
    b = fixed.shape[0]
    d, h, lanes = fixed.shape[2], fixed.shape[3], fixed.shape[4]
    bd = _choose_bd(d, h)
    nchunk = d // bd

    na = predicted_affine.shape[1]
    pa = jnp.pad(predicted_affine[:, None, :], ((0, 0), (0, 0), (0, 128 - na)))
    ta = jnp.pad(true_affine[:, None, :], ((0, 0), (0, 0), (0, 128 - na)))

    out = pl.pallas_call(
        functools.partial(_loss_kernel, nchunk=nchunk),
        grid=(b, 2, nchunk),
        in_specs=[
            pl.BlockSpec((1, 1, 128), lambda bb, p, c: (bb, 0, 0)),
            pl.BlockSpec((1, 1, 128), lambda bb, p, c: (bb, 0, 0)),
            pl.BlockSpec((1, 1, bd, h, lanes),
                         lambda bb, p, c: (bb, 0, c, 0, 0)),
            pl.BlockSpec((1, 1, bd, h, lanes),
                         lambda bb, p, c: (bb, 0, c, 0, 0)),
        ],
        out_specs=pl.BlockSpec((1, 1, 128), lambda bb, p, c: (bb, 0, 0)),
        out_shape=jax.ShapeDtypeStruct((b, 1, 128), jnp.float32),
        scratch_shapes=[
            pltpu.SMEM((4,), jnp.float32),
            pltpu.VMEM((2, _BINS, _BINS), jnp.float32),
        ],
        compiler_params=pltpu.CompilerParams(
            dimension_semantics=("parallel", "arbitrary", "arbitrary")),
    )(pa, ta, fixed, warped)

    affine_loss = jnp.sum(out[:, 0, 0]) / (b * na)
    sim_loss = -jnp.mean(out[:, 0, 1])
    return affine_loss + sim_loss


# speedup vs baseline: 1.9044x; 1.0025x over previous
import functools

import jax
import jax.numpy as jnp
from jax.experimental import pallas as pl
from jax.experimental.pallas import tpu as pltpu

_BINS = 64
_EPS = 1e-10
_SUB_R = 32


def _loss_kernel(pa_ref, ta_ref, f_ref, w_ref, out_ref, mm_ref, acc_ref,
                 *, nchunk):
    phase = pl.program_id(1)
    c = pl.program_id(2)

    @pl.when(jnp.logical_and(phase == 0, c == 0))
    def _init_mm():
        mm_ref[0] = jnp.float32(jnp.inf)
        mm_ref[1] = jnp.float32(-jnp.inf)
        mm_ref[2] = jnp.float32(jnp.inf)
        mm_ref[3] = jnp.float32(-jnp.inf)

    @pl.when(phase == 0)
    def _minmax():
        fb = f_ref[0, 0]
        wb = w_ref[0, 0]
        mm_ref[0] = jnp.minimum(mm_ref[0], jnp.min(fb))
        mm_ref[1] = jnp.maximum(mm_ref[1], jnp.max(fb))
        mm_ref[2] = jnp.minimum(mm_ref[2], jnp.min(wb))
        mm_ref[3] = jnp.maximum(mm_ref[3], jnp.max(wb))

    @pl.when(phase == 1)
    def _hist():
        @pl.when(c == 0)
        def _zero_acc():
            acc_ref[...] = jnp.zeros_like(acc_ref)


        fmin = mm_ref[0]
        wmin = mm_ref[2]
        inv_f = 1.0 / (mm_ref[1] - fmin + _EPS)
        inv_w = 1.0 / (mm_ref[3] - wmin + _EPS)

        bd, hh, lanes = f_ref.shape[2], f_ref.shape[3], f_ref.shape[4]
        bins2d = jax.lax.broadcasted_iota(
            jnp.int32, (_BINS, lanes), 0).astype(jnp.int8)
        f8t = jnp.float8_e4m3fn

        def make_body(dd):
            def body(i, _):
                f8 = f_ref[0, 0, dd, pl.ds(i * _SUB_R, _SUB_R), :]
                w8 = w_ref[0, 0, dd, pl.ds(i * _SUB_R, _SUB_R), :]
                fi8 = jnp.clip(jnp.floor((f8 - fmin) * inv_f * (_BINS - 1)),
                               0.0, _BINS - 1.0).astype(jnp.int8)
                wi8 = jnp.clip(jnp.floor((w8 - wmin) * inv_w * (_BINS - 1)),
                               0.0, _BINS - 1.0).astype(jnp.int8)
                accs = [acc_ref[k] for k in range(4)]
                for j in range(_SUB_R):
                    fr = jax.lax.slice(fi8, (j, 0), (j + 1, lanes))
                    wr = jax.lax.slice(wi8, (j, 0), (j + 1, lanes))
                    ohf = jnp.where(bins2d == fr, f8t(1), f8t(0))
                    ohw = jnp.where(bins2d == wr, f8t(1), f8t(0))
                    dot = jax.lax.dot_general(
                        ohf, ohw, (((1,), (1,)), ((), ())),
                        preferred_element_type=jnp.float32)
                    accs[j % 4] = dot + accs[j % 4]
                for k in range(4):
                    acc_ref[k] = accs[k]
                return 0
            return body

        for dd in range(bd):
            jax.lax.fori_loop(0, hh // _SUB_R, make_body(dd), 0)

        @pl.when(c == nchunk - 1)
        def _finalize():
            counts = ((acc_ref[0] + acc_ref[1])
                      + (acc_ref[2] + acc_ref[3]))
            total = jnp.sum(counts)
            inv_n = 1.0 / (total + _EPS)
            joint = counts * inv_n
            p_f = jnp.sum(joint, axis=1, keepdims=True)
            p_w = jnp.sum(joint, axis=0, keepdims=True)
            h_f = -jnp.sum(p_f * jnp.log(p_f + _EPS))
            h_w = -jnp.sum(p_w * jnp.log(p_w + _EPS))
            h_j = -jnp.sum(joint * jnp.log(joint + _EPS))
            mi = h_f + h_w - h_j
            nmi = 2.0 * mi / (h_f + h_w + _EPS)
            d = pa_ref[0] - ta_ref[0]
            sumsq = jnp.sum(d * d)
            lane = jax.lax.broadcasted_iota(jnp.int32, (1, 128), 1)
            out_ref[0] = jnp.where(lane == 0, sumsq,
                                   jnp.where(lane == 1, nmi, 0.0))


def _choose_bd(d, h):
    for bd in (6, 4, 3, 2, 1):
        if d % bd == 0 and h % _SUB_R == 0:
            return bd
    return 1


@jax.jit
def kernel(predicted_affine, true_affine, fixed, warped):
    b = fixed.shape[0]
    d, h, lanes = fixed.shape[2], fixed.shape[3], fixed.shape[4]
    bd = _choose_bd(d, h)
    nchunk = d // bd

    na = predicted_affine.shape[1]
    pa = jnp.pad(predicted_affine[:, None, :], ((0, 0), (0, 0), (0, 128 - na)))
    ta = jnp.pad(true_affine[:, None, :], ((0, 0), (0, 0), (0, 128 - na)))

    out = pl.pallas_call(
        functools.partial(_loss_kernel, nchunk=nchunk),
        grid=(b, 2, nchunk),
        in_specs=[
            pl.BlockSpec((1, 1, 128), lambda bb, p, c: (bb, 0, 0)),
            pl.BlockSpec((1, 1, 128), lambda bb, p, c: (bb, 0, 0)),
            pl.BlockSpec((1, 1, bd, h, lanes),
                         lambda bb, p, c: (bb, 0, c, 0, 0)),
            pl.BlockSpec((1, 1, bd, h, lanes),
                         lambda bb, p, c: (bb, 0, c, 0, 0)),
        ],
        out_specs=pl.BlockSpec((1, 1, 128), lambda bb, p, c: (bb, 0, 0)),
        out_shape=jax.ShapeDtypeStruct((b, 1, 128), jnp.float32),
        scratch_shapes=[
            pltpu.SMEM((4,), jnp.float32),
            pltpu.VMEM((4, _BINS, _BINS), jnp.float32),
        ],
        compiler_params=pltpu.CompilerParams(
            dimension_semantics=("parallel", "arbitrary", "arbitrary")),
    )(pa, ta, fixed, warped)

    affine_loss = jnp.sum(out[:, 0, 0]) / (b * na)
    sim_loss = -jnp.mean(out[:, 0, 1])
    return affine_loss + sim_loss
